# Initial kernel scaffold; baseline (speedup 1.0000x reference)
import jax
import jax.numpy as jnp
from jax.experimental import pallas as pl


def kernel(x, condition_idx, embeddings):
    raise NotImplementedError("write your pallas kernel here")


# speedup vs baseline: 1.3174x; 1.3174x over previous
import functools

import jax
import jax.numpy as jnp
from jax import lax
from jax.experimental import pallas as pl
from jax.experimental.pallas import tpu as pltpu
from jax.experimental.pallas import tpu_sc as plsc

B = 16384
D = 128
NC = 2
NS = 16
NW = NC * NS
B_PER_W = B // NW
R = 256
N_CHUNKS = B_PER_W // R

_mesh = plsc.VectorSubcoreMesh(core_axis_name="c", subcore_axis_name="s")


@functools.partial(
    pl.kernel,
    mesh=_mesh,
    out_type=jax.ShapeDtypeStruct((B, D), jnp.float32),
    scratch_types=[
        pltpu.VMEM((R,), jnp.int32),
        pltpu.VMEM((R, D), jnp.float32),
        pltpu.VMEM((R, D), jnp.float32),
        pltpu.SemaphoreType.DMA,
        pltpu.SemaphoreType.DMA,
    ],
)
def _sc_embed_add(x_hbm, idx_hbm, emb_hbm, out_hbm, idx_v, x_v, rows_v,
                  sem_g, sem_x):
    wid = lax.axis_index("s") * NC + lax.axis_index("c")
    base = wid * B_PER_W
    for ch in range(N_CHUNKS):
        row0 = base + ch * R
        pltpu.sync_copy(idx_hbm.at[pl.ds(row0, R)], idx_v)
        g = pltpu.async_copy(emb_hbm.at[idx_v], rows_v, sem_g)
        xc = pltpu.async_copy(x_hbm.at[pl.ds(row0, R)], x_v, sem_x)
        g.wait()
        xc.wait()

        def add_row(r, carry):
            for j in range(D // 16):
                sl = pl.ds(j * 16, 16)
                x_v[r, sl] = x_v[r, sl] + rows_v[r, sl]
            return carry

        lax.fori_loop(0, R, add_row, 0)
        pltpu.sync_copy(x_v, out_hbm.at[pl.ds(row0, R)])


def kernel(x, condition_idx, embeddings):
    idx = condition_idx.astype(jnp.int32)
    return _sc_embed_add(x, idx, embeddings)
